# baseline (speedup 1.0000x reference)
import functools

import jax
import jax.numpy as jnp
from jax import lax
from jax.experimental import pallas as pl
from jax.experimental.pallas import tpu as pltpu
from jax.experimental.pallas import tpu_sc as plsc

_MESH = plsc.VectorSubcoreMesh(core_axis_name="c", subcore_axis_name="s")


@functools.partial(
    pl.kernel,
    out_type=jax.ShapeDtypeStruct((2,), jnp.float32),
    mesh=_MESH,
    scratch_types=[pltpu.VMEM((16,), jnp.float32)],
)
def _gather_head(x_hbm, out_hbm, buf):
    cid = lax.axis_index("c")
    sid = lax.axis_index("s")

    @pl.when(jnp.logical_and(cid == 0, sid == 0))
    def _():
        pltpu.sync_copy(x_hbm.at[pl.ds(0, 16)], buf)
        pltpu.sync_copy(buf.at[pl.ds(0, 2)], out_hbm)


def kernel(x):
    return _gather_head(x)


# speedup vs baseline: 1.0865x; 1.0865x over previous
import functools

import jax
import jax.numpy as jnp
from jax import lax
from jax.experimental import pallas as pl
from jax.experimental.pallas import tpu as pltpu
from jax.experimental.pallas import tpu_sc as plsc

_MESH = plsc.VectorSubcoreMesh(
    core_axis_name="c", subcore_axis_name="s", num_cores=1, num_subcores=1
)


@functools.partial(
    pl.kernel,
    out_type=jax.ShapeDtypeStruct((2,), jnp.float32),
    mesh=_MESH,
    scratch_types=[pltpu.VMEM((16,), jnp.float32)],
)
def _gather_head(x_hbm, out_hbm, buf):
    pltpu.sync_copy(x_hbm.at[pl.ds(0, 16)], buf)
    pltpu.sync_copy(buf.at[pl.ds(0, 2)], out_hbm)


def kernel(x):
    return _gather_head(x)


# speedup vs baseline: 16.4572x; 15.1469x over previous
import jax
import jax.numpy as jnp
from jax.experimental import pallas as pl


def _gather_head_body(x_ref, o_ref):
    o_ref[...] = x_ref[pl.ds(0, 2)]


def kernel(x):
    return pl.pallas_call(
        _gather_head_body,
        grid=(1,),
        in_specs=[pl.BlockSpec((128,), lambda i: (0,))],
        out_specs=pl.BlockSpec((2,), lambda i: (0,)),
        out_shape=jax.ShapeDtypeStruct((2,), jnp.float32),
    )(x)


# speedup vs baseline: 16.5385x; 1.0049x over previous
import jax
import jax.numpy as jnp
from jax.experimental import pallas as pl
from jax.experimental.pallas import tpu as pltpu


def _gather_head_body(x_ref, o_ref, buf, sem):
    cp = pltpu.make_async_copy(x_ref.at[pl.ds(0, 128)], buf, sem)
    cp.start()
    cp.wait()
    o_ref[...] = buf[pl.ds(0, 2)]


def kernel(x):
    return pl.pallas_call(
        _gather_head_body,
        in_specs=[pl.BlockSpec(memory_space=pl.ANY)],
        out_specs=pl.BlockSpec(memory_space=pltpu.MemorySpace.VMEM),
        out_shape=jax.ShapeDtypeStruct((2,), jnp.float32),
        scratch_shapes=[pltpu.VMEM((128,), jnp.float32), pltpu.SemaphoreType.DMA],
    )(x)
